# Initial kernel scaffold; baseline (speedup 1.0000x reference)
import jax
import jax.numpy as jnp
from jax.experimental import pallas as pl


def kernel(pnl):
    raise NotImplementedError("write your pallas kernel here")


# speedup vs baseline: 12.2143x; 12.2143x over previous
import functools

import jax
import jax.numpy as jnp
from jax import lax
from jax.experimental import pallas as pl
from jax.experimental.pallas import tpu as pltpu
from jax.experimental.pallas import tpu_sc as plsc

N = 1048576
K = 52428
NC = 2
NS = 16
NW = NC * NS
CHUNK = N // NW
GROUPS = CHUNK // 16
B1 = 2048
B2 = 2048
B3 = 1024

_mesh = plsc.VectorSubcoreMesh(core_axis_name="c", subcore_axis_name="s")
_params = pltpu.CompilerParams(needs_layout_passes=False)


def _monokeys(x):
    b = lax.bitcast_convert_type(x, jnp.int32)
    ks = jnp.where(b < 0, b ^ jnp.int32(0x7FFFFFFF), b)
    return ks ^ jnp.int32(-2147483648)


def _shr(v, amt):
    return lax.shift_right_logical(v, jnp.full((16,), amt, jnp.int32))


def _zero_ref(ref, nwords, dtype):
    z = jnp.zeros((16,), dtype)

    def body(i, c):
        ref[pl.ds(i * 16, 16)] = z
        return c

    lax.fori_loop(0, nwords // 16, body, 0)


def _wid():
    return lax.axis_index("s") * NC + lax.axis_index("c")


@functools.partial(
    pl.kernel,
    out_type=jax.ShapeDtypeStruct((NW, B1), jnp.int32),
    mesh=_mesh,
    compiler_params=_params,
    scratch_types=[
        pltpu.VMEM((CHUNK,), jnp.float32),
        pltpu.VMEM((B1,), jnp.int32),
    ],
)
def _hist1(pnl_hbm, out_hbm, data_v, hist_v):
    w = _wid()
    pltpu.sync_copy(pnl_hbm.at[pl.ds(w * CHUNK, CHUNK)], data_v)
    _zero_ref(hist_v, B1, jnp.int32)
    ones = jnp.ones((16,), jnp.int32)

    def body(i, c):
        x = data_v[pl.ds(i * 16, 16)]
        ku = _monokeys(x)
        bkt = _shr(ku, 21)
        plsc.addupdate_scatter(hist_v, [bkt], ones)
        return c

    lax.fori_loop(0, GROUPS, body, 0)
    pltpu.sync_copy(hist_v, out_hbm.at[w])


@functools.partial(
    pl.kernel,
    out_type=jax.ShapeDtypeStruct((NW, B2), jnp.int32),
    mesh=_mesh,
    compiler_params=_params,
    scratch_types=[
        pltpu.VMEM((CHUNK,), jnp.float32),
        pltpu.VMEM((128,), jnp.int32),
        pltpu.VMEM((B2,), jnp.int32),
    ],
)
def _hist2(pnl_hbm, pref_hbm, out_hbm, data_v, pref_v, hist_v):
    w = _wid()
    pltpu.sync_copy(pnl_hbm.at[pl.ds(w * CHUNK, CHUNK)], data_v)
    pltpu.sync_copy(pref_hbm, pref_v)
    _zero_ref(hist_v, B2, jnp.int32)
    ones = jnp.ones((16,), jnp.int32)
    p1 = pref_v[pl.ds(0, 16)]

    def body(i, c):
        x = data_v[pl.ds(i * 16, 16)]
        ku = _monokeys(x)
        match = _shr(ku, 21) == p1
        bkt = _shr(ku, 10) & jnp.int32(B2 - 1)
        plsc.addupdate_scatter(hist_v, [bkt], ones, mask=match)
        return c

    lax.fori_loop(0, GROUPS, body, 0)
    pltpu.sync_copy(hist_v, out_hbm.at[w])


@functools.partial(
    pl.kernel,
    out_type=(
        jax.ShapeDtypeStruct((NW, B3), jnp.int32),
        jax.ShapeDtypeStruct((NW, B3), jnp.float32),
        jax.ShapeDtypeStruct((NW, 128), jnp.float32),
    ),
    mesh=_mesh,
    compiler_params=_params,
    scratch_types=[
        pltpu.VMEM((CHUNK,), jnp.float32),
        pltpu.VMEM((128,), jnp.int32),
        pltpu.VMEM((B3,), jnp.int32),
        pltpu.VMEM((B3,), jnp.float32),
        pltpu.VMEM((128,), jnp.float32),
    ],
)
def _hist3(pnl_hbm, pref_hbm, cnt_hbm, sum_hbm, below_hbm,
           data_v, pref_v, cnt_v, sum_v, below_v):
    w = _wid()
    pltpu.sync_copy(pnl_hbm.at[pl.ds(w * CHUNK, CHUNK)], data_v)
    pltpu.sync_copy(pref_hbm, pref_v)
    _zero_ref(cnt_v, B3, jnp.int32)
    _zero_ref(sum_v, B3, jnp.float32)
    ones = jnp.ones((16,), jnp.int32)
    fz = jnp.zeros((16,), jnp.float32)
    p2 = pref_v[pl.ds(0, 16)]

    def body(i, acc):
        x = data_v[pl.ds(i * 16, 16)]
        ku = _monokeys(x)
        hi = _shr(ku, 10)
        match = hi == p2
        below = hi < p2
        bkt = ku & jnp.int32(B3 - 1)
        plsc.addupdate_scatter(cnt_v, [bkt], ones, mask=match)
        plsc.addupdate_scatter(sum_v, [bkt], x, mask=match)
        return acc + jnp.where(below, x, fz)

    acc = lax.fori_loop(0, GROUPS, body, fz)
    pltpu.sync_copy(cnt_v, cnt_hbm.at[w])
    pltpu.sync_copy(sum_v, sum_hbm.at[w])
    _zero_ref(below_v, 128, jnp.float32)
    below_v[pl.ds(0, 16)] = acc
    pltpu.sync_copy(below_v, below_hbm.at[w])


def _splat16(v):
    return jnp.full((128,), 1, jnp.int32) * v


def kernel(pnl):
    h1 = _hist1(pnl)
    c1 = jnp.sum(h1, axis=0)
    cum1 = jnp.cumsum(c1)
    sel1 = cum1 < K
    b1 = jnp.sum(sel1).astype(jnp.int32)
    cb1 = jnp.sum(jnp.where(sel1, c1, 0))
    k2 = K - cb1

    h2 = _hist2(pnl, _splat16(b1))
    c2 = jnp.sum(h2, axis=0)
    cum2 = jnp.cumsum(c2)
    sel2 = cum2 < k2
    b2 = jnp.sum(sel2).astype(jnp.int32)
    cb2 = jnp.sum(jnp.where(sel2, c2, 0))
    k3 = k2 - cb2

    p2 = (b1 << 11) | b2
    h3, s3, below = _hist3(pnl, _splat16(p2))
    c3 = jnp.sum(h3, axis=0)
    s3 = jnp.sum(s3, axis=0)
    cum3 = jnp.cumsum(c3)
    sel3 = cum3 < k3
    b3 = jnp.sum(sel3).astype(jnp.int32)
    cb3 = jnp.sum(jnp.where(sel3, c3, 0))
    in_bucket_sum = jnp.sum(jnp.where(sel3, s3, 0.0))

    key = ((b1.astype(jnp.uint32) << 21)
           | (b2.astype(jnp.uint32) << 10)
           | b3.astype(jnp.uint32))
    bits = jnp.where(key >= jnp.uint32(2147483648),
                     key ^ jnp.uint32(2147483648), ~key)
    t = lax.bitcast_convert_type(bits, jnp.float32)

    count_below = cb1 + cb2 + cb3
    sum_below = jnp.sum(below) + in_bucket_sum
    cvar = (sum_below + (K - count_below).astype(jnp.float32) * t) / K
    return -cvar


# speedup vs baseline: 12.4554x; 1.0197x over previous
import functools

import jax
import jax.numpy as jnp
from jax import lax
from jax.experimental import pallas as pl
from jax.experimental.pallas import tpu as pltpu
from jax.experimental.pallas import tpu_sc as plsc

N = 1048576
K = 52428
NC = 2
NS = 16
NW = NC * NS
CHUNK = N // NW
GROUPS = CHUNK // 16
UNROLL = 8
B1 = 2048
B2 = 2048
B3 = 1024

_mesh = plsc.VectorSubcoreMesh(core_axis_name="c", subcore_axis_name="s")
_params = pltpu.CompilerParams(needs_layout_passes=False)


def _monokeys(x):
    b = lax.bitcast_convert_type(x, jnp.int32)
    m = lax.shift_right_arithmetic(b, jnp.full((16,), 31, jnp.int32))
    return b ^ (m | jnp.int32(-2147483648))


def _shr(v, amt):
    return lax.shift_right_logical(v, jnp.full((16,), amt, jnp.int32))


def _zero_ref(ref, nwords, dtype):
    z = jnp.zeros((16,), dtype)

    def body(i, c):
        ref[pl.ds(i * 16, 16)] = z
        return c

    lax.fori_loop(0, nwords // 16, body, 0)


def _wid():
    return lax.axis_index("s") * NC + lax.axis_index("c")


@functools.partial(
    pl.kernel,
    out_type=jax.ShapeDtypeStruct((NW, B1), jnp.int32),
    mesh=_mesh,
    compiler_params=_params,
    scratch_types=[
        pltpu.VMEM((CHUNK,), jnp.float32),
        pltpu.VMEM((B1,), jnp.int32),
    ],
)
def _hist1(pnl_hbm, out_hbm, data_v, hist_v):
    w = _wid()
    pltpu.sync_copy(pnl_hbm.at[pl.ds(w * CHUNK, CHUNK)], data_v)
    _zero_ref(hist_v, B1, jnp.int32)
    ones = jnp.ones((16,), jnp.int32)

    def body(i, c):
        base = i * (16 * UNROLL)
        for u in range(UNROLL):
            x = data_v[pl.ds(base + u * 16, 16)]
            ku = _monokeys(x)
            bkt = _shr(ku, 21)
            plsc.addupdate_scatter(hist_v, [bkt], ones)
        return c

    lax.fori_loop(0, GROUPS // UNROLL, body, 0)
    pltpu.sync_copy(hist_v, out_hbm.at[w])


@functools.partial(
    pl.kernel,
    out_type=jax.ShapeDtypeStruct((NW, B2), jnp.int32),
    mesh=_mesh,
    compiler_params=_params,
    scratch_types=[
        pltpu.VMEM((CHUNK,), jnp.float32),
        pltpu.VMEM((128,), jnp.int32),
        pltpu.VMEM((B2,), jnp.int32),
    ],
)
def _hist2(pnl_hbm, pref_hbm, out_hbm, data_v, pref_v, hist_v):
    w = _wid()
    pltpu.sync_copy(pnl_hbm.at[pl.ds(w * CHUNK, CHUNK)], data_v)
    pltpu.sync_copy(pref_hbm, pref_v)
    _zero_ref(hist_v, B2, jnp.int32)
    ones = jnp.ones((16,), jnp.int32)
    p1 = pref_v[pl.ds(0, 16)]

    def body(i, c):
        base = i * (16 * UNROLL)
        for u in range(UNROLL):
            x = data_v[pl.ds(base + u * 16, 16)]
            ku = _monokeys(x)
            match = _shr(ku, 21) == p1
            bkt = _shr(ku, 10) & jnp.int32(B2 - 1)
            plsc.addupdate_scatter(hist_v, [bkt], ones, mask=match)
        return c

    lax.fori_loop(0, GROUPS // UNROLL, body, 0)
    pltpu.sync_copy(hist_v, out_hbm.at[w])


@functools.partial(
    pl.kernel,
    out_type=(
        jax.ShapeDtypeStruct((NW, B3), jnp.int32),
        jax.ShapeDtypeStruct((NW, B3), jnp.float32),
        jax.ShapeDtypeStruct((NW, 128), jnp.float32),
    ),
    mesh=_mesh,
    compiler_params=_params,
    scratch_types=[
        pltpu.VMEM((CHUNK,), jnp.float32),
        pltpu.VMEM((128,), jnp.int32),
        pltpu.VMEM((B3,), jnp.int32),
        pltpu.VMEM((B3,), jnp.float32),
        pltpu.VMEM((128,), jnp.float32),
    ],
)
def _hist3(pnl_hbm, pref_hbm, cnt_hbm, sum_hbm, below_hbm,
           data_v, pref_v, cnt_v, sum_v, below_v):
    w = _wid()
    pltpu.sync_copy(pnl_hbm.at[pl.ds(w * CHUNK, CHUNK)], data_v)
    pltpu.sync_copy(pref_hbm, pref_v)
    _zero_ref(cnt_v, B3, jnp.int32)
    _zero_ref(sum_v, B3, jnp.float32)
    ones = jnp.ones((16,), jnp.int32)
    fz = jnp.zeros((16,), jnp.float32)
    p2 = pref_v[pl.ds(0, 16)]

    def body(i, acc):
        base = i * (16 * UNROLL)
        for u in range(UNROLL):
            x = data_v[pl.ds(base + u * 16, 16)]
            ku = _monokeys(x)
            hi = _shr(ku, 10)
            match = hi == p2
            below = hi < p2
            bkt = ku & jnp.int32(B3 - 1)
            plsc.addupdate_scatter(cnt_v, [bkt], ones, mask=match)
            plsc.addupdate_scatter(sum_v, [bkt], x, mask=match)
            acc = acc + jnp.where(below, x, fz)
        return acc

    acc = lax.fori_loop(0, GROUPS // UNROLL, body, fz)
    pltpu.sync_copy(cnt_v, cnt_hbm.at[w])
    pltpu.sync_copy(sum_v, sum_hbm.at[w])
    _zero_ref(below_v, 128, jnp.float32)
    below_v[pl.ds(0, 16)] = acc
    pltpu.sync_copy(below_v, below_hbm.at[w])


def _splat16(v):
    return jnp.full((128,), 1, jnp.int32) * v


def kernel(pnl):
    h1 = _hist1(pnl)
    c1 = jnp.sum(h1, axis=0)
    cum1 = jnp.cumsum(c1)
    sel1 = cum1 < K
    b1 = jnp.sum(sel1).astype(jnp.int32)
    cb1 = jnp.sum(jnp.where(sel1, c1, 0))
    k2 = K - cb1

    h2 = _hist2(pnl, _splat16(b1))
    c2 = jnp.sum(h2, axis=0)
    cum2 = jnp.cumsum(c2)
    sel2 = cum2 < k2
    b2 = jnp.sum(sel2).astype(jnp.int32)
    cb2 = jnp.sum(jnp.where(sel2, c2, 0))
    k3 = k2 - cb2

    p2 = (b1 << 11) | b2
    h3, s3, below = _hist3(pnl, _splat16(p2))
    c3 = jnp.sum(h3, axis=0)
    s3 = jnp.sum(s3, axis=0)
    cum3 = jnp.cumsum(c3)
    sel3 = cum3 < k3
    b3 = jnp.sum(sel3).astype(jnp.int32)
    cb3 = jnp.sum(jnp.where(sel3, c3, 0))
    in_bucket_sum = jnp.sum(jnp.where(sel3, s3, 0.0))

    key = ((b1.astype(jnp.uint32) << 21)
           | (b2.astype(jnp.uint32) << 10)
           | b3.astype(jnp.uint32))
    bits = jnp.where(key >= jnp.uint32(2147483648),
                     key ^ jnp.uint32(2147483648), ~key)
    t = lax.bitcast_convert_type(bits, jnp.float32)

    count_below = cb1 + cb2 + cb3
    sum_below = jnp.sum(below) + in_bucket_sum
    cvar = (sum_below + (K - count_below).astype(jnp.float32) * t) / K
    return -cvar
